# Initial kernel scaffold; baseline (speedup 1.0000x reference)
import jax
import jax.numpy as jnp
from jax.experimental import pallas as pl


def kernel(x_u, x_v, x_y, index_u, index_v, index_y, att_w_u, att_b_u, att_w_v, att_b_v, att_w_y, att_b_y, net_w, net_b):
    raise NotImplementedError("write your pallas kernel here")


# speedup vs baseline: 2.5803x; 2.5803x over previous
import functools

import jax
import jax.numpy as jnp
from jax.experimental import pallas as pl
from jax.experimental.pallas import tpu as pltpu

N = 100000
D = 128
S = 256
CHUNK = 512
NCHUNK = (N + CHUNK - 1) // CHUNK
NPAD = NCHUNK * CHUNK


def _body(x_ref, ids_ref, aw_ref, ab_ref, nw_ref, nb_ref, out_ref, acc, cnt):
    p = pl.program_id(0)
    i = pl.program_id(1)

    @pl.when((p == 0) & (i == 0))
    def _init():
        acc[...] = jnp.zeros_like(acc)
        cnt[...] = jnp.zeros_like(cnt)

    x = x_ref[0]
    ids = ids_ref[0]
    w = aw_ref[0]
    b = ab_ref[0][0, 0]

    z = jnp.sum(x * w, axis=1, keepdims=True) + b
    a = jax.nn.sigmoid(z)
    y = a * x

    seg = jax.lax.broadcasted_iota(jnp.int32, (S, CHUNK), 0).astype(jnp.float32)
    m = jnp.where(seg == ids, 1.0, 0.0)
    acc[p] += jnp.dot(m, y, preferred_element_type=jnp.float32)
    cnt[p] += jnp.sum(m, axis=1, keepdims=True)

    @pl.when((p == 2) & (i == NCHUNK - 1))
    def _fin():
        e = acc[...] / jnp.maximum(cnt[...], 1.0)
        ecat = jnp.concatenate([e[0], e[1], e[2]], axis=1)
        out_ref[...] = (
            jnp.dot(ecat, nw_ref[...], preferred_element_type=jnp.float32)
            + nb_ref[...]
        )


@functools.partial(jax.jit, static_argnames=("interpret",))
def _run(x_all, ids_all, aw_all, ab_all, net_w, net_b, interpret=False):
    grid = (3, NCHUNK)
    return pl.pallas_call(
        _body,
        grid=grid,
        in_specs=[
            pl.BlockSpec((1, CHUNK, D), lambda p, i: (p, i, 0)),
            pl.BlockSpec((1, 1, CHUNK), lambda p, i: (p * NCHUNK + i, 0, 0)),
            pl.BlockSpec((1, 1, D), lambda p, i: (p, 0, 0)),
            pl.BlockSpec((1, 1, D), lambda p, i: (p, 0, 0)),
            pl.BlockSpec((3 * D, D), lambda p, i: (0, 0)),
            pl.BlockSpec((1, D), lambda p, i: (0, 0)),
        ],
        out_specs=pl.BlockSpec((S, D), lambda p, i: (0, 0)),
        out_shape=jax.ShapeDtypeStruct((S, D), jnp.float32),
        scratch_shapes=[
            pltpu.VMEM((3, S, D), jnp.float32),
            pltpu.VMEM((3, S, D), jnp.float32),
        ],
        interpret=interpret,
    )(x_all, ids_all, aw_all, ab_all, net_w, net_b)


def kernel(x_u, x_v, x_y, index_u, index_v, index_y,
           att_w_u, att_b_u, att_w_v, att_b_v, att_w_y, att_b_y,
           net_w, net_b):
    pad = NPAD - N
    x_all = jnp.stack([x_u, x_v, x_y])
    x_all = jnp.pad(x_all, ((0, 0), (0, pad), (0, 0)))
    ids = jnp.stack([index_u, index_v, index_y]).astype(jnp.float32)
    ids = jnp.pad(ids, ((0, 0), (0, pad)), constant_values=float(2 * S))
    ids_all = ids.reshape(3 * NCHUNK, 1, CHUNK)
    aw_all = jnp.stack([att_w_u, att_w_v, att_w_y]).reshape(3, 1, D)
    ab_all = jnp.stack([att_b_u, att_b_v, att_b_y]).reshape(3, 1, 1)
    ab_all = jnp.broadcast_to(ab_all, (3, 1, D))
    return _run(x_all, ids_all, aw_all, ab_all, net_w,
                net_b.reshape(1, D))
